# baseline (speedup 1.0000x reference)
import functools

import jax
import jax.numpy as jnp
from jax import lax
from jax.experimental import pallas as pl
from jax.experimental.pallas import tpu as pltpu
from jax.experimental.pallas import tpu_sc as plsc

TOP_K = 2
CAP = 192

NC = 2
NS = 16
NW = NC * NS
LANES = 16

INVALID = 1 << 30


def _router_body(x_ref, gw_ref, w_ref, gs_ref, gg_ref, xb_ref, cnt_ref):
    tb = x_ref.shape[0]
    n_e = gw_ref.shape[0]

    @pl.when(pl.program_id(0) == 0)
    def _():
        cnt_ref[...] = jnp.zeros_like(cnt_ref)

    x = x_ref[...]
    dh = x_ref.shape[1] // 2

    def _rne(v):
        iv = lax.bitcast_convert_type(v, jnp.int32)
        rnd = ((iv >> 16) & 1) + 0x7FFF
        return iv + rnd

    ia = _rne(x[:, :dh])
    ib = _rne(x[:, dh:])
    xb_ref[...] = lax.shift_right_logical(ia, 16) | (
        ib & jnp.int32(-65536))
    gw = gw_ref[...]
    logits = lax.dot_general(x, gw, (((1,), (1,)), ((), ())),
                             preferred_element_type=jnp.float32)
    eidx = lax.broadcasted_iota(jnp.int32, (tb, n_e), 1)
    m1 = jnp.max(logits, axis=1, keepdims=True)
    e1 = jnp.min(jnp.where(logits == m1, eidx, n_e), axis=1, keepdims=True)
    l2 = jnp.where(eidx == e1, jnp.float32(-1e30), logits)
    m2 = jnp.max(l2, axis=1, keepdims=True)
    e2 = jnp.min(jnp.where(l2 == m2, eidx, n_e), axis=1, keepdims=True)
    w1 = jax.nn.sigmoid(m1 - m2)
    w2 = jax.nn.sigmoid(m2 - m1)

    hit = ((eidx == e1) | (eidx == e2)).astype(jnp.float32)
    ri = lax.broadcasted_iota(jnp.int32, (tb, tb), 0)
    ci = lax.broadcasted_iota(jnp.int32, (tb, tb), 1)
    ltri = (ci < ri).astype(jnp.float32)
    pos = lax.dot_general(ltri, hit, (((1,), (0,)), ((), ())),
                          preferred_element_type=jnp.float32) + cnt_ref[...]
    cnt_ref[...] = cnt_ref[...] + jnp.sum(hit, axis=0, keepdims=True)

    pos1 = jnp.sum(jnp.where(eidx == e1, pos, 0.0), axis=1,
                   keepdims=True).astype(jnp.int32)
    pos2 = jnp.sum(jnp.where(eidx == e2, pos, 0.0), axis=1,
                   keepdims=True).astype(jnp.int32)
    v1 = pos1 < CAP
    v2 = pos2 < CAP
    g1 = e1 * CAP + pos1
    g2 = e2 * CAP + pos2
    w_ref[...] = jnp.concatenate(
        [jnp.where(v1, w1, 0.0), jnp.where(v2, w2, 0.0)], axis=1)
    gs_ref[...] = jnp.concatenate(
        [jnp.where(v1, g1, INVALID), jnp.where(v2, g2, INVALID)], axis=1)
    gg_ref[...] = jnp.concatenate(
        [jnp.where(v1, g1, 0), jnp.where(v2, g2, 0)], axis=1)


def _router(x, gate_w):
    t, d = x.shape
    n_e = gate_w.shape[0]
    tb = 256
    grid = (t // tb,)
    return pl.pallas_call(
        _router_body,
        grid=grid,
        in_specs=[
            pl.BlockSpec((tb, d), lambda i: (i, 0)),
            pl.BlockSpec((n_e, d), lambda i: (0, 0)),
        ],
        out_specs=[
            pl.BlockSpec((tb, TOP_K), lambda i: (i, 0)),
            pl.BlockSpec((tb, TOP_K), lambda i: (i, 0)),
            pl.BlockSpec((tb, TOP_K), lambda i: (i, 0)),
            pl.BlockSpec((tb, d // 2), lambda i: (i, 0)),
        ],
        out_shape=[
            jax.ShapeDtypeStruct((t, TOP_K), jnp.float32),
            jax.ShapeDtypeStruct((t, TOP_K), jnp.int32),
            jax.ShapeDtypeStruct((t, TOP_K), jnp.int32),
            jax.ShapeDtypeStruct((t, d // 2), jnp.int32),
        ],
        scratch_shapes=[pltpu.VMEM((1, n_e), jnp.float32)],
    )(x, gate_w)


def _make_dispatch(t, d, n_e):
    slots = n_e * CAP
    xs_rows = slots + CAP
    toks_per_w = t // NW
    mesh = plsc.VectorSubcoreMesh(core_axis_name="c", subcore_axis_name="s")

    @functools.partial(
        pl.kernel,
        out_type=jax.ShapeDtypeStruct((xs_rows, d), jnp.int32),
        mesh=mesh,
        scratch_types=[
            pltpu.VMEM((toks_per_w, d), jnp.int32),
            pltpu.VMEM((toks_per_w,), jnp.int32),
            pltpu.VMEM((toks_per_w,), jnp.int32),
            pltpu.SemaphoreType.DMA,
        ],
        compiler_params=pltpu.CompilerParams(needs_layout_passes=False),
    )
    def dispatch(gs_hbm, xb_hbm, xs_hbm, rows_v, dst0_v, dst1_v, sem):
        wid = lax.axis_index("s") * NC + lax.axis_index("c")
        tok0 = wid * toks_per_w
        c_rows = pltpu.async_copy(xb_hbm.at[pl.ds(tok0, toks_per_w)],
                                  rows_v, sem)
        pltpu.sync_copy(gs_hbm.at[0, pl.ds(tok0, toks_per_w)], dst0_v)
        pltpu.sync_copy(gs_hbm.at[1, pl.ds(tok0, toks_per_w)], dst1_v)
        trash = slots + wid
        for m in range(toks_per_w // LANES):
            sl = pl.ds(m * LANES, LANES)
            v0 = dst0_v[sl]
            dst0_v[sl] = jnp.where(v0 < slots, v0, trash)
            v1 = dst1_v[sl]
            dst1_v[sl] = jnp.where(v1 < slots, v1, trash)
        c_rows.wait()
        c0 = pltpu.async_copy(rows_v, xs_hbm.at[dst0_v], sem)
        c1 = pltpu.async_copy(rows_v, xs_hbm.at[dst1_v], sem)
        c0.wait()
        c1.wait()

    return dispatch


def _expert_body(xs_ref, wgu_ref, wd_ref, ys_ref):
    ff = wd_ref.shape[1]
    xp = xs_ref[...]
    xa = lax.bitcast_convert_type(lax.shift_left(xp, 16), jnp.float32)
    xb = lax.bitcast_convert_type(xp & jnp.int32(-65536), jnp.float32)
    xs = jnp.concatenate([xa, xb], axis=1).astype(jnp.bfloat16)
    wgu = wgu_ref[0].astype(jnp.bfloat16)
    gu = lax.dot_general(xs, wgu, (((1,), (0,)), ((), ())),
                         preferred_element_type=jnp.float32)
    gate = gu[:, :ff]
    up = gu[:, ff:]
    h = (up * (gate * jax.nn.sigmoid(gate))).astype(jnp.bfloat16)
    wd = wd_ref[0].astype(jnp.bfloat16)
    ys_ref[...] = lax.dot_general(h, wd, (((1,), (0,)), ((), ())),
                                  preferred_element_type=jnp.float32)


def _experts(xs, w_gate_up, w_down):
    n_e, d, dff2 = w_gate_up.shape
    ff = w_down.shape[1]
    return pl.pallas_call(
        _expert_body,
        grid=(n_e,),
        in_specs=[
            pl.BlockSpec((CAP, d // 2), lambda e: (e, 0)),
            pl.BlockSpec((1, d, dff2), lambda e: (e, 0, 0)),
            pl.BlockSpec((1, ff, d), lambda e: (e, 0, 0)),
        ],
        out_specs=pl.BlockSpec((CAP, d), lambda e: (e, 0)),
        out_shape=jax.ShapeDtypeStruct((n_e * CAP, d), jnp.float32),
    )(xs, w_gate_up, w_down)


def _make_combine(t, d, n_e):
    toks_per_w = t // NW
    tchunk = 16
    n_chunks = toks_per_w // tchunk
    nd = d // LANES
    mesh = plsc.VectorSubcoreMesh(core_axis_name="c", subcore_axis_name="s")

    @functools.partial(
        pl.kernel,
        out_type=jax.ShapeDtypeStruct((t, d), jnp.float32),
        mesh=mesh,
        scratch_types=[
            pltpu.VMEM((toks_per_w,), jnp.float32),
            pltpu.VMEM((toks_per_w,), jnp.float32),
            pltpu.VMEM((toks_per_w,), jnp.int32),
            pltpu.VMEM((toks_per_w,), jnp.int32),
            pltpu.VMEM((2, tchunk, d), jnp.float32),
            pltpu.VMEM((2, tchunk, d), jnp.float32),
            pltpu.VMEM((2, tchunk, d), jnp.float32),
            pltpu.SemaphoreType.DMA,
            pltpu.SemaphoreType.DMA,
        ],
        compiler_params=pltpu.CompilerParams(needs_layout_passes=False),
    )
    def combine(ys_hbm, gg_hbm, w_hbm, out_hbm, w0_v, w1_v, i0_v, i1_v,
                r0_v, r1_v, out_v, semg, semw):
        wid = lax.axis_index("s") * NC + lax.axis_index("c")
        tok0 = wid * toks_per_w
        pltpu.sync_copy(w_hbm.at[0, pl.ds(tok0, toks_per_w)], w0_v)
        pltpu.sync_copy(w_hbm.at[1, pl.ds(tok0, toks_per_w)], w1_v)
        pltpu.sync_copy(gg_hbm.at[0, pl.ds(tok0, toks_per_w)], i0_v)
        pltpu.sync_copy(gg_hbm.at[1, pl.ds(tok0, toks_per_w)], i1_v)

        def start_gather(j):
            b = j % 2
            c0 = pltpu.async_copy(
                ys_hbm.at[i0_v.at[pl.ds(j * tchunk, tchunk)]],
                r0_v.at[b], semg)
            c1 = pltpu.async_copy(
                ys_hbm.at[i1_v.at[pl.ds(j * tchunk, tchunk)]],
                r1_v.at[b], semg)
            return c0, c1

        gathers = {0: start_gather(0)}
        writes = {}
        for j in range(n_chunks):
            b = j % 2
            c0, c1 = gathers.pop(j)
            c0.wait()
            c1.wait()
            if j + 1 < n_chunks:
                gathers[j + 1] = start_gather(j + 1)
            if j >= 2:
                writes.pop(j - 2).wait()

            def row_body(i, c2, j=j, b=b):
                ia = jnp.full((LANES,), j * tchunk + i, jnp.int32)
                wa = plsc.load_gather(w0_v, [ia])
                wb = plsc.load_gather(w1_v, [ia])
                for s in range(nd):
                    sl = pl.ds(s * LANES, LANES)
                    out_v[b, i, sl] = (r0_v[b, i, sl] * wa
                                       + r1_v[b, i, sl] * wb)
                return c2

            lax.fori_loop(0, tchunk, row_body, 0)
            writes[j] = pltpu.async_copy(
                out_v.at[b], out_hbm.at[pl.ds(tok0 + j * tchunk, tchunk)],
                semw)
        for j in sorted(writes):
            writes.pop(j).wait()

    return combine


def kernel(hidden_states, gate_w, w_gate_up, w_down):
    b, s, d = hidden_states.shape
    t = b * s
    n_e = gate_w.shape[0]
    x = hidden_states.reshape(t, d)

    w_sel, gs, gg, xb = _router(x, gate_w)
    gs_t = gs.T.reshape(TOP_K, t)
    gg_t = gg.T.reshape(TOP_K, t)
    w_t = w_sel.T.reshape(TOP_K, t)
    xs = _make_dispatch(t, d // 2, n_e)(gs_t, xb)
    ys = _experts(xs, w_gate_up, w_down)
    out = _make_combine(t, d, n_e)(ys, gg_t, w_t)
    return out.reshape(b, s, d)


# speedup vs baseline: 1.0148x; 1.0148x over previous
import functools

import jax
import jax.numpy as jnp
from jax import lax
from jax.experimental import pallas as pl
from jax.experimental.pallas import tpu as pltpu
from jax.experimental.pallas import tpu_sc as plsc

TOP_K = 2
CAP = 192

NC = 2
NS = 16
NW = NC * NS
LANES = 16

INVALID = 1 << 30


def _router_body(x_ref, gw_ref, w_ref, gs_ref, gg_ref, xb_ref, cnt_ref):
    tb = x_ref.shape[0]
    n_e = gw_ref.shape[0]

    @pl.when(pl.program_id(0) == 0)
    def _():
        cnt_ref[...] = jnp.zeros_like(cnt_ref)

    x = x_ref[...]
    dh = x_ref.shape[1] // 2

    def _rne(v):
        iv = lax.bitcast_convert_type(v, jnp.int32)
        rnd = ((iv >> 16) & 1) + 0x7FFF
        return iv + rnd

    ia = _rne(x[:, :dh])
    ib = _rne(x[:, dh:])
    xb_ref[...] = lax.shift_right_logical(ia, 16) | (
        ib & jnp.int32(-65536))
    gw = gw_ref[...]
    logits = lax.dot_general(x, gw, (((1,), (1,)), ((), ())),
                             preferred_element_type=jnp.float32)
    eidx = lax.broadcasted_iota(jnp.int32, (tb, n_e), 1)
    m1 = jnp.max(logits, axis=1, keepdims=True)
    e1 = jnp.min(jnp.where(logits == m1, eidx, n_e), axis=1, keepdims=True)
    l2 = jnp.where(eidx == e1, jnp.float32(-1e30), logits)
    m2 = jnp.max(l2, axis=1, keepdims=True)
    e2 = jnp.min(jnp.where(l2 == m2, eidx, n_e), axis=1, keepdims=True)
    w1 = jax.nn.sigmoid(m1 - m2)
    w2 = jax.nn.sigmoid(m2 - m1)

    hit = ((eidx == e1) | (eidx == e2)).astype(jnp.float32)
    ri = lax.broadcasted_iota(jnp.int32, (tb, tb), 0)
    ci = lax.broadcasted_iota(jnp.int32, (tb, tb), 1)
    ltri = (ci < ri).astype(jnp.float32)
    pos = lax.dot_general(ltri, hit, (((1,), (0,)), ((), ())),
                          preferred_element_type=jnp.float32) + cnt_ref[...]
    cnt_ref[...] = cnt_ref[...] + jnp.sum(hit, axis=0, keepdims=True)

    pos1 = jnp.sum(jnp.where(eidx == e1, pos, 0.0), axis=1,
                   keepdims=True).astype(jnp.int32)
    pos2 = jnp.sum(jnp.where(eidx == e2, pos, 0.0), axis=1,
                   keepdims=True).astype(jnp.int32)
    v1 = pos1 < CAP
    v2 = pos2 < CAP
    g1 = e1 * CAP + pos1
    g2 = e2 * CAP + pos2
    w_ref[...] = jnp.concatenate(
        [jnp.where(v1, w1, 0.0), jnp.where(v2, w2, 0.0)], axis=1)
    gs_ref[...] = jnp.concatenate(
        [jnp.where(v1, g1, INVALID), jnp.where(v2, g2, INVALID)], axis=1)
    gg_ref[...] = jnp.concatenate(
        [jnp.where(v1, g1, 0), jnp.where(v2, g2, 0)], axis=1)


def _router(x, gate_w):
    t, d = x.shape
    n_e = gate_w.shape[0]
    tb = 512
    grid = (t // tb,)
    return pl.pallas_call(
        _router_body,
        grid=grid,
        in_specs=[
            pl.BlockSpec((tb, d), lambda i: (i, 0)),
            pl.BlockSpec((n_e, d), lambda i: (0, 0)),
        ],
        out_specs=[
            pl.BlockSpec((tb, TOP_K), lambda i: (i, 0)),
            pl.BlockSpec((tb, TOP_K), lambda i: (i, 0)),
            pl.BlockSpec((tb, TOP_K), lambda i: (i, 0)),
            pl.BlockSpec((tb, d // 2), lambda i: (i, 0)),
        ],
        out_shape=[
            jax.ShapeDtypeStruct((t, TOP_K), jnp.float32),
            jax.ShapeDtypeStruct((t, TOP_K), jnp.int32),
            jax.ShapeDtypeStruct((t, TOP_K), jnp.int32),
            jax.ShapeDtypeStruct((t, d // 2), jnp.int32),
        ],
        scratch_shapes=[pltpu.VMEM((1, n_e), jnp.float32)],
    )(x, gate_w)


def _make_dispatch(t, d, n_e):
    slots = n_e * CAP
    xs_rows = slots + CAP
    toks_per_w = t // NW
    mesh = plsc.VectorSubcoreMesh(core_axis_name="c", subcore_axis_name="s")

    @functools.partial(
        pl.kernel,
        out_type=jax.ShapeDtypeStruct((xs_rows, d), jnp.int32),
        mesh=mesh,
        scratch_types=[
            pltpu.VMEM((toks_per_w, d), jnp.int32),
            pltpu.VMEM((toks_per_w,), jnp.int32),
            pltpu.VMEM((toks_per_w,), jnp.int32),
            pltpu.SemaphoreType.DMA,
        ],
        compiler_params=pltpu.CompilerParams(needs_layout_passes=False),
    )
    def dispatch(gs_hbm, xb_hbm, xs_hbm, rows_v, dst0_v, dst1_v, sem):
        wid = lax.axis_index("s") * NC + lax.axis_index("c")
        tok0 = wid * toks_per_w
        c_rows = pltpu.async_copy(xb_hbm.at[pl.ds(tok0, toks_per_w)],
                                  rows_v, sem)
        pltpu.sync_copy(gs_hbm.at[0, pl.ds(tok0, toks_per_w)], dst0_v)
        pltpu.sync_copy(gs_hbm.at[1, pl.ds(tok0, toks_per_w)], dst1_v)
        trash = slots + wid
        for m in range(toks_per_w // LANES):
            sl = pl.ds(m * LANES, LANES)
            v0 = dst0_v[sl]
            dst0_v[sl] = jnp.where(v0 < slots, v0, trash)
            v1 = dst1_v[sl]
            dst1_v[sl] = jnp.where(v1 < slots, v1, trash)
        c_rows.wait()
        c0 = pltpu.async_copy(rows_v, xs_hbm.at[dst0_v], sem)
        c1 = pltpu.async_copy(rows_v, xs_hbm.at[dst1_v], sem)
        c0.wait()
        c1.wait()

    return dispatch


def _expert_body(xs_ref, wgu_ref, wd_ref, ys_ref):
    ff = wd_ref.shape[1]
    xp = xs_ref[...]
    xa = lax.bitcast_convert_type(lax.shift_left(xp, 16), jnp.float32)
    xb = lax.bitcast_convert_type(xp & jnp.int32(-65536), jnp.float32)
    xs = jnp.concatenate([xa, xb], axis=1).astype(jnp.bfloat16)
    wgu = wgu_ref[0].astype(jnp.bfloat16)
    gu = lax.dot_general(xs, wgu, (((1,), (0,)), ((), ())),
                         preferred_element_type=jnp.float32)
    gate = gu[:, :ff]
    up = gu[:, ff:]
    h = (up * (gate * jax.nn.sigmoid(gate))).astype(jnp.bfloat16)
    wd = wd_ref[0].astype(jnp.bfloat16)
    ys_ref[...] = lax.dot_general(h, wd, (((1,), (0,)), ((), ())),
                                  preferred_element_type=jnp.float32)


def _experts(xs, w_gate_up, w_down):
    n_e, d, dff2 = w_gate_up.shape
    ff = w_down.shape[1]
    return pl.pallas_call(
        _expert_body,
        grid=(n_e,),
        in_specs=[
            pl.BlockSpec((CAP, d // 2), lambda e: (e, 0)),
            pl.BlockSpec((1, d, dff2), lambda e: (e, 0, 0)),
            pl.BlockSpec((1, ff, d), lambda e: (e, 0, 0)),
        ],
        out_specs=pl.BlockSpec((CAP, d), lambda e: (e, 0)),
        out_shape=jax.ShapeDtypeStruct((n_e * CAP, d), jnp.float32),
    )(xs, w_gate_up, w_down)


def _make_combine(t, d, n_e):
    toks_per_w = t // NW
    tchunk = 16
    n_chunks = toks_per_w // tchunk
    nd = d // LANES
    mesh = plsc.VectorSubcoreMesh(core_axis_name="c", subcore_axis_name="s")

    @functools.partial(
        pl.kernel,
        out_type=jax.ShapeDtypeStruct((t, d), jnp.float32),
        mesh=mesh,
        scratch_types=[
            pltpu.VMEM((toks_per_w,), jnp.float32),
            pltpu.VMEM((toks_per_w,), jnp.float32),
            pltpu.VMEM((toks_per_w,), jnp.int32),
            pltpu.VMEM((toks_per_w,), jnp.int32),
            pltpu.VMEM((2, tchunk, d), jnp.float32),
            pltpu.VMEM((2, tchunk, d), jnp.float32),
            pltpu.VMEM((2, tchunk, d), jnp.float32),
            pltpu.SemaphoreType.DMA,
            pltpu.SemaphoreType.DMA,
        ],
        compiler_params=pltpu.CompilerParams(needs_layout_passes=False),
    )
    def combine(ys_hbm, gg_hbm, w_hbm, out_hbm, w0_v, w1_v, i0_v, i1_v,
                r0_v, r1_v, out_v, semg, semw):
        wid = lax.axis_index("s") * NC + lax.axis_index("c")
        tok0 = wid * toks_per_w
        pltpu.sync_copy(w_hbm.at[0, pl.ds(tok0, toks_per_w)], w0_v)
        pltpu.sync_copy(w_hbm.at[1, pl.ds(tok0, toks_per_w)], w1_v)
        pltpu.sync_copy(gg_hbm.at[0, pl.ds(tok0, toks_per_w)], i0_v)
        pltpu.sync_copy(gg_hbm.at[1, pl.ds(tok0, toks_per_w)], i1_v)

        def start_gather(j):
            b = j % 2
            c0 = pltpu.async_copy(
                ys_hbm.at[i0_v.at[pl.ds(j * tchunk, tchunk)]],
                r0_v.at[b], semg)
            c1 = pltpu.async_copy(
                ys_hbm.at[i1_v.at[pl.ds(j * tchunk, tchunk)]],
                r1_v.at[b], semg)
            return c0, c1

        gathers = {0: start_gather(0)}
        writes = {}
        for j in range(n_chunks):
            b = j % 2
            c0, c1 = gathers.pop(j)
            c0.wait()
            c1.wait()
            if j + 1 < n_chunks:
                gathers[j + 1] = start_gather(j + 1)
            if j >= 2:
                writes.pop(j - 2).wait()

            def row_body(i, c2, j=j, b=b):
                ia = jnp.full((LANES,), j * tchunk + i, jnp.int32)
                wa = plsc.load_gather(w0_v, [ia])
                wb = plsc.load_gather(w1_v, [ia])
                for s in range(nd):
                    sl = pl.ds(s * LANES, LANES)
                    out_v[b, i, sl] = (r0_v[b, i, sl] * wa
                                       + r1_v[b, i, sl] * wb)
                return c2

            lax.fori_loop(0, tchunk, row_body, 0)
            writes[j] = pltpu.async_copy(
                out_v.at[b], out_hbm.at[pl.ds(tok0 + j * tchunk, tchunk)],
                semw)
        for j in sorted(writes):
            writes.pop(j).wait()

    return combine


def kernel(hidden_states, gate_w, w_gate_up, w_down):
    b, s, d = hidden_states.shape
    t = b * s
    n_e = gate_w.shape[0]
    x = hidden_states.reshape(t, d)

    w_sel, gs, gg, xb = _router(x, gate_w)
    gs_t = gs.T.reshape(TOP_K, t)
    gg_t = gg.T.reshape(TOP_K, t)
    w_t = w_sel.T.reshape(TOP_K, t)
    xs = _make_dispatch(t, d // 2, n_e)(gs_t, xb)
    ys = _experts(xs, w_gate_up, w_down)
    out = _make_combine(t, d, n_e)(ys, gg_t, w_t)
    return out.reshape(b, s, d)


# speedup vs baseline: 1.0162x; 1.0015x over previous
import functools

import jax
import jax.numpy as jnp
from jax import lax
from jax.experimental import pallas as pl
from jax.experimental.pallas import tpu as pltpu
from jax.experimental.pallas import tpu_sc as plsc

TOP_K = 2
CAP = 192

NC = 2
NS = 16
NW = NC * NS
LANES = 16

INVALID = 1 << 30


def _router_body(x_ref, gw_ref, w_ref, gs_ref, gg_ref, xb_ref, cnt_ref):
    tb = x_ref.shape[0]
    n_e = gw_ref.shape[0]

    @pl.when(pl.program_id(0) == 0)
    def _():
        cnt_ref[...] = jnp.zeros_like(cnt_ref)

    x = x_ref[...]
    dh = x_ref.shape[1] // 2

    def _rne(v):
        iv = lax.bitcast_convert_type(v, jnp.int32)
        rnd = ((iv >> 16) & 1) + 0x7FFF
        return iv + rnd

    ia = _rne(x[:, :dh])
    ib = _rne(x[:, dh:])
    xb_ref[...] = lax.shift_right_logical(ia, 16) | (
        ib & jnp.int32(-65536))
    gw = gw_ref[...]
    logits = lax.dot_general(x, gw, (((1,), (1,)), ((), ())),
                             preferred_element_type=jnp.float32)
    eidx = lax.broadcasted_iota(jnp.int32, (tb, n_e), 1)
    m1 = jnp.max(logits, axis=1, keepdims=True)
    e1 = jnp.min(jnp.where(logits == m1, eidx, n_e), axis=1, keepdims=True)
    l2 = jnp.where(eidx == e1, jnp.float32(-1e30), logits)
    m2 = jnp.max(l2, axis=1, keepdims=True)
    e2 = jnp.min(jnp.where(l2 == m2, eidx, n_e), axis=1, keepdims=True)
    w1 = jax.nn.sigmoid(m1 - m2)
    w2 = jax.nn.sigmoid(m2 - m1)

    hit = ((eidx == e1) | (eidx == e2)).astype(jnp.float32)
    ri = lax.broadcasted_iota(jnp.int32, (tb, tb), 0)
    ci = lax.broadcasted_iota(jnp.int32, (tb, tb), 1)
    ltri = (ci < ri).astype(jnp.float32)
    pos = lax.dot_general(ltri, hit, (((1,), (0,)), ((), ())),
                          preferred_element_type=jnp.float32) + cnt_ref[...]
    cnt_ref[...] = cnt_ref[...] + jnp.sum(hit, axis=0, keepdims=True)

    pos1 = jnp.sum(jnp.where(eidx == e1, pos, 0.0), axis=1,
                   keepdims=True).astype(jnp.int32)
    pos2 = jnp.sum(jnp.where(eidx == e2, pos, 0.0), axis=1,
                   keepdims=True).astype(jnp.int32)
    v1 = pos1 < CAP
    v2 = pos2 < CAP
    g1 = e1 * CAP + pos1
    g2 = e2 * CAP + pos2
    w_ref[...] = jnp.concatenate(
        [jnp.where(v1, w1, 0.0), jnp.where(v2, w2, 0.0)], axis=1)
    gs_ref[...] = jnp.concatenate(
        [jnp.where(v1, g1, INVALID), jnp.where(v2, g2, INVALID)], axis=1)
    gg_ref[...] = jnp.concatenate(
        [jnp.where(v1, g1, 0), jnp.where(v2, g2, 0)], axis=1)


def _router(x, gate_w):
    t, d = x.shape
    n_e = gate_w.shape[0]
    tb = 1024
    grid = (t // tb,)
    return pl.pallas_call(
        _router_body,
        grid=grid,
        in_specs=[
            pl.BlockSpec((tb, d), lambda i: (i, 0)),
            pl.BlockSpec((n_e, d), lambda i: (0, 0)),
        ],
        out_specs=[
            pl.BlockSpec((tb, TOP_K), lambda i: (i, 0)),
            pl.BlockSpec((tb, TOP_K), lambda i: (i, 0)),
            pl.BlockSpec((tb, TOP_K), lambda i: (i, 0)),
            pl.BlockSpec((tb, d // 2), lambda i: (i, 0)),
        ],
        out_shape=[
            jax.ShapeDtypeStruct((t, TOP_K), jnp.float32),
            jax.ShapeDtypeStruct((t, TOP_K), jnp.int32),
            jax.ShapeDtypeStruct((t, TOP_K), jnp.int32),
            jax.ShapeDtypeStruct((t, d // 2), jnp.int32),
        ],
        scratch_shapes=[pltpu.VMEM((1, n_e), jnp.float32)],
    )(x, gate_w)


def _make_dispatch(t, d, n_e):
    slots = n_e * CAP
    xs_rows = slots + CAP
    toks_per_w = t // NW
    mesh = plsc.VectorSubcoreMesh(core_axis_name="c", subcore_axis_name="s")

    @functools.partial(
        pl.kernel,
        out_type=jax.ShapeDtypeStruct((xs_rows, d), jnp.int32),
        mesh=mesh,
        scratch_types=[
            pltpu.VMEM((toks_per_w, d), jnp.int32),
            pltpu.VMEM((toks_per_w,), jnp.int32),
            pltpu.VMEM((toks_per_w,), jnp.int32),
            pltpu.SemaphoreType.DMA,
        ],
        compiler_params=pltpu.CompilerParams(needs_layout_passes=False),
    )
    def dispatch(gs_hbm, xb_hbm, xs_hbm, rows_v, dst0_v, dst1_v, sem):
        wid = lax.axis_index("s") * NC + lax.axis_index("c")
        tok0 = wid * toks_per_w
        c_rows = pltpu.async_copy(xb_hbm.at[pl.ds(tok0, toks_per_w)],
                                  rows_v, sem)
        pltpu.sync_copy(gs_hbm.at[0, pl.ds(tok0, toks_per_w)], dst0_v)
        pltpu.sync_copy(gs_hbm.at[1, pl.ds(tok0, toks_per_w)], dst1_v)
        trash = slots + wid
        for m in range(toks_per_w // LANES):
            sl = pl.ds(m * LANES, LANES)
            v0 = dst0_v[sl]
            dst0_v[sl] = jnp.where(v0 < slots, v0, trash)
            v1 = dst1_v[sl]
            dst1_v[sl] = jnp.where(v1 < slots, v1, trash)
        c_rows.wait()
        c0 = pltpu.async_copy(rows_v, xs_hbm.at[dst0_v], sem)
        c1 = pltpu.async_copy(rows_v, xs_hbm.at[dst1_v], sem)
        c0.wait()
        c1.wait()

    return dispatch


def _expert_body(xs_ref, wgu_ref, wd_ref, ys_ref):
    ff = wd_ref.shape[1]
    xp = xs_ref[...]
    xa = lax.bitcast_convert_type(lax.shift_left(xp, 16), jnp.float32)
    xb = lax.bitcast_convert_type(xp & jnp.int32(-65536), jnp.float32)
    xs = jnp.concatenate([xa, xb], axis=1).astype(jnp.bfloat16)
    wgu = wgu_ref[0].astype(jnp.bfloat16)
    gu = lax.dot_general(xs, wgu, (((1,), (0,)), ((), ())),
                         preferred_element_type=jnp.float32)
    gate = gu[:, :ff]
    up = gu[:, ff:]
    h = (up * (gate * jax.nn.sigmoid(gate))).astype(jnp.bfloat16)
    wd = wd_ref[0].astype(jnp.bfloat16)
    ys_ref[...] = lax.dot_general(h, wd, (((1,), (0,)), ((), ())),
                                  preferred_element_type=jnp.float32)


def _experts(xs, w_gate_up, w_down):
    n_e, d, dff2 = w_gate_up.shape
    ff = w_down.shape[1]
    return pl.pallas_call(
        _expert_body,
        grid=(n_e,),
        in_specs=[
            pl.BlockSpec((CAP, d // 2), lambda e: (e, 0)),
            pl.BlockSpec((1, d, dff2), lambda e: (e, 0, 0)),
            pl.BlockSpec((1, ff, d), lambda e: (e, 0, 0)),
        ],
        out_specs=pl.BlockSpec((CAP, d), lambda e: (e, 0)),
        out_shape=jax.ShapeDtypeStruct((n_e * CAP, d), jnp.float32),
    )(xs, w_gate_up, w_down)


def _make_combine(t, d, n_e):
    toks_per_w = t // NW
    tchunk = 16
    n_chunks = toks_per_w // tchunk
    nd = d // LANES
    mesh = plsc.VectorSubcoreMesh(core_axis_name="c", subcore_axis_name="s")

    @functools.partial(
        pl.kernel,
        out_type=jax.ShapeDtypeStruct((t, d), jnp.float32),
        mesh=mesh,
        scratch_types=[
            pltpu.VMEM((toks_per_w,), jnp.float32),
            pltpu.VMEM((toks_per_w,), jnp.float32),
            pltpu.VMEM((toks_per_w,), jnp.int32),
            pltpu.VMEM((toks_per_w,), jnp.int32),
            pltpu.VMEM((2, tchunk, d), jnp.float32),
            pltpu.VMEM((2, tchunk, d), jnp.float32),
            pltpu.VMEM((2, tchunk, d), jnp.float32),
            pltpu.SemaphoreType.DMA,
            pltpu.SemaphoreType.DMA,
        ],
        compiler_params=pltpu.CompilerParams(needs_layout_passes=False),
    )
    def combine(ys_hbm, gg_hbm, w_hbm, out_hbm, w0_v, w1_v, i0_v, i1_v,
                r0_v, r1_v, out_v, semg, semw):
        wid = lax.axis_index("s") * NC + lax.axis_index("c")
        tok0 = wid * toks_per_w
        pltpu.sync_copy(w_hbm.at[0, pl.ds(tok0, toks_per_w)], w0_v)
        pltpu.sync_copy(w_hbm.at[1, pl.ds(tok0, toks_per_w)], w1_v)
        pltpu.sync_copy(gg_hbm.at[0, pl.ds(tok0, toks_per_w)], i0_v)
        pltpu.sync_copy(gg_hbm.at[1, pl.ds(tok0, toks_per_w)], i1_v)

        def start_gather(j):
            b = j % 2
            c0 = pltpu.async_copy(
                ys_hbm.at[i0_v.at[pl.ds(j * tchunk, tchunk)]],
                r0_v.at[b], semg)
            c1 = pltpu.async_copy(
                ys_hbm.at[i1_v.at[pl.ds(j * tchunk, tchunk)]],
                r1_v.at[b], semg)
            return c0, c1

        gathers = {0: start_gather(0)}
        writes = {}
        for j in range(n_chunks):
            b = j % 2
            c0, c1 = gathers.pop(j)
            c0.wait()
            c1.wait()
            if j + 1 < n_chunks:
                gathers[j + 1] = start_gather(j + 1)
            if j >= 2:
                writes.pop(j - 2).wait()

            def row_body(i, c2, j=j, b=b):
                ia = jnp.full((LANES,), j * tchunk + i, jnp.int32)
                wa = plsc.load_gather(w0_v, [ia])
                wb = plsc.load_gather(w1_v, [ia])
                for s in range(nd):
                    sl = pl.ds(s * LANES, LANES)
                    out_v[b, i, sl] = (r0_v[b, i, sl] * wa
                                       + r1_v[b, i, sl] * wb)
                return c2

            lax.fori_loop(0, tchunk, row_body, 0)
            writes[j] = pltpu.async_copy(
                out_v.at[b], out_hbm.at[pl.ds(tok0 + j * tchunk, tchunk)],
                semw)
        for j in sorted(writes):
            writes.pop(j).wait()

    return combine


def kernel(hidden_states, gate_w, w_gate_up, w_down):
    b, s, d = hidden_states.shape
    t = b * s
    n_e = gate_w.shape[0]
    x = hidden_states.reshape(t, d)

    w_sel, gs, gg, xb = _router(x, gate_w)
    gs_t = gs.T.reshape(TOP_K, t)
    gg_t = gg.T.reshape(TOP_K, t)
    w_t = w_sel.T.reshape(TOP_K, t)
    xs = _make_dispatch(t, d // 2, n_e)(gs_t, xb)
    ys = _experts(xs, w_gate_up, w_down)
    out = _make_combine(t, d, n_e)(ys, gg_t, w_t)
    return out.reshape(b, s, d)
